# baseline (speedup 1.0000x reference)
import functools

import jax
import jax.numpy as jnp
from jax import lax
from jax.experimental import pallas as pl
from jax.experimental.pallas import tpu as pltpu
from jax.experimental.pallas import tpu_sc as plsc

_NC, _NS = 2, 16
_NW = _NC * _NS
_CHUNK = 40
_LANES = 16


def _fuse_tables_body(pe_ref, a_ref, b_ref, out_ref):
    out_ref[0] = jnp.dot(pe_ref[...], a_ref[0],
                         preferred_element_type=jnp.float32) + b_ref[...]


_HP = 256


def _make_tables(pe_table, W, b):
    P, H = pe_table.shape
    A = W.reshape(H, 4, H).transpose(1, 2, 0)
    A = jnp.pad(A, ((0, 0), (0, 0), (0, _HP - H)))
    bq = jnp.pad((0.25 * b).reshape(1, H), ((0, 0), (0, _HP - H)))
    T = pl.pallas_call(
        _fuse_tables_body,
        grid=(4,),
        in_specs=[
            pl.BlockSpec((P, H), lambda k: (0, 0)),
            pl.BlockSpec((1, H, _HP), lambda k: (k, 0, 0)),
            pl.BlockSpec((1, _HP), lambda k: (0, 0)),
        ],
        out_specs=pl.BlockSpec((1, P, _HP), lambda k: (k, 0, 0)),
        out_shape=jax.ShapeDtypeStruct((4, P, _HP), jnp.float32),
    )(pe_table.astype(jnp.float32), A.astype(jnp.float32),
      bq.astype(jnp.float32))
    return T.reshape(4 * P, _HP)


def _make_sc_lookup(n_rows, H, n_tab):
    per_w = n_rows // _NW
    nch = per_w // _CHUNK
    vec = H // _LANES
    cw = (_CHUNK // 8) * 8 * _HP
    mesh = plsc.VectorSubcoreMesh(core_axis_name="c", subcore_axis_name="s")

    @functools.partial(
        pl.kernel,
        mesh=mesh,
        out_type=jax.ShapeDtypeStruct((n_rows * _HP,), jnp.float32),
        scratch_types=[
            pltpu.VMEM((2, 4, _CHUNK), jnp.int32),
            pltpu.VMEM((2, 4, _CHUNK, H), jnp.float32),
            pltpu.VMEM((2, cw), jnp.float32),
            pltpu.VMEM_SHARED((n_tab, H), jnp.float32),
            pltpu.SemaphoreType.DMA,
            pltpu.SemaphoreType.DMA,
            pltpu.SemaphoreType.DMA,
            pltpu.SemaphoreType.DMA,
            pltpu.SemaphoreType.DMA,
            pltpu.SemaphoreType.DMA,
        ],
        compiler_params=pltpu.CompilerParams(use_tc_tiling_on_sc=False),
    )
    def sc_fn(t_hbm, i0_hbm, i1_hbm, i2_hbm, i3_hbm, out_hbm,
              idx_v, rows_v, out_v, t_sp, si0, si1, sg0, sg1, ss0, ss1):
        wid = lax.axis_index("s") * _NC + lax.axis_index("c")
        base = wid * nch

        @pl.when(lax.axis_index("s") == 0)
        def _():
            pltpu.sync_copy(t_hbm.at[:, pl.ds(0, H)], t_sp)

        plsc.subcore_barrier()
        idx_refs = (i0_hbm, i1_hbm, i2_hbm, i3_hbm)
        sem_i = (si0, si1)
        sem_g = (sg0, sg1)
        sem_s = (ss0, ss1)

        def start_idx(ci, slot):
            off = (base + ci) * _CHUNK
            for k in range(4):
                pltpu.async_copy(idx_refs[k].at[pl.ds(off, _CHUNK)],
                                 idx_v.at[slot, k], sem_i[slot])

        def wait_idx(slot):
            pltpu.make_async_copy(i0_hbm.at[pl.ds(0, 4 * _CHUNK)],
                                  idx_v.at[slot], sem_i[slot]).wait()

        def start_gathers(buf):
            for k in range(4):
                pltpu.async_copy(t_sp.at[idx_v.at[buf, k]],
                                 rows_v.at[buf, k], sem_g[buf])

        def wait_gathers(buf):
            pltpu.make_async_copy(out_hbm.at[pl.ds(0, 4 * cw)],
                                  rows_v.at[buf], sem_g[buf]).wait()

        def wait_scatter(buf):
            pltpu.make_async_copy(out_hbm.at[pl.ds(0, cw)],
                                  out_v.at[buf], sem_s[buf]).wait()

        def compute_and_scatter(ci, buf):
            def row_body(r, c2):
                rblk = (r // 8) * (8 * _HP) + (r % 8) * 128
                for v in range(vec):
                    sl = pl.ds(v * _LANES, _LANES)
                    acc = (rows_v[buf, 0, r, sl] + rows_v[buf, 1, r, sl]) + (
                        rows_v[buf, 2, r, sl] + rows_v[buf, 3, r, sl])
                    off = rblk + (v // 8) * 1024 + (v % 8) * _LANES
                    out_v[buf, pl.ds(off, _LANES)] = jnp.maximum(acc, 0.0)
                return c2

            lax.fori_loop(0, _CHUNK, row_body, 0)
            pltpu.async_copy(out_v.at[buf],
                             out_hbm.at[pl.ds((base + ci) * cw, cw)],
                             sem_s[buf])

        def step(ci, buf, first):
            @pl.when(ci + 1 < nch)
            def _():
                wait_idx(1 - buf)
                start_gathers(1 - buf)
            wait_gathers(buf)

            @pl.when(ci + 2 < nch)
            def _():
                start_idx(ci + 2, buf)

            if not first:
                @pl.when(ci >= 2)
                def _():
                    wait_scatter(buf)
            compute_and_scatter(ci, buf)

        start_idx(0, 0)
        start_idx(1, 1)
        wait_idx(0)
        start_gathers(0)
        step(0, 0, True)

        def chunk_body(h, carry):
            step(1 + 2 * h, 1, False)
            step(2 + 2 * h, 0, False)
            return carry

        lax.fori_loop(0, (nch - 1) // 2, chunk_body, 0)
        if (nch - 1) % 2 == 1:
            step(nch - 1, (nch - 1) % 2, False)
        wait_scatter(0)
        wait_scatter(1)

    return sc_fn


def kernel(pos_s, pos_e, pe_table, W, b):
    B, L = pos_s.shape
    P, H = pe_table.shape
    M = P // 2
    n = B * L * L
    T = _make_tables(pe_table, W, b)
    ps = pos_s.astype(jnp.int32)
    pe = pos_e.astype(jnp.int32)

    def rel(a, c, off):
        d = jnp.clip(a[:, :, None] - c[:, None, :] + M, 0, P - 1)
        return (d + off).reshape(-1)

    out = _make_sc_lookup(n, H, 4 * P)(
        T, rel(ps, ps, 0), rel(ps, pe, P), rel(pe, ps, 2 * P),
        rel(pe, pe, 3 * P))
    out = out.reshape(B, L, L // 8, 2, 8, 128).transpose(0, 1, 2, 4, 3, 5)
    out = out.reshape(B, L, L, _HP)[..., :H]
    return jnp.maximum(out, 0.0)


# speedup vs baseline: 1.1578x; 1.1578x over previous
import functools

import jax
import jax.numpy as jnp
from jax import lax
from jax.experimental import pallas as pl
from jax.experimental.pallas import tpu as pltpu
from jax.experimental.pallas import tpu_sc as plsc

_NC, _NS = 2, 16
_NW = _NC * _NS
_CHUNK = 40
_LANES = 16


def _fuse_tables_body(pe_ref, a_ref, b_ref, out_ref):
    out_ref[0] = jnp.dot(pe_ref[...], a_ref[0],
                         preferred_element_type=jnp.float32) + b_ref[...]


_HP = 256


def _make_tables(pe_table, W, b):
    P, H = pe_table.shape
    A = W.reshape(H, 4, H).transpose(1, 2, 0)
    A = jnp.pad(A, ((0, 0), (0, 0), (0, _HP - H)))
    bq = jnp.pad((0.25 * b).reshape(1, H), ((0, 0), (0, _HP - H)))
    T = pl.pallas_call(
        _fuse_tables_body,
        grid=(4,),
        in_specs=[
            pl.BlockSpec((P, H), lambda k: (0, 0)),
            pl.BlockSpec((1, H, _HP), lambda k: (k, 0, 0)),
            pl.BlockSpec((1, _HP), lambda k: (0, 0)),
        ],
        out_specs=pl.BlockSpec((1, P, _HP), lambda k: (k, 0, 0)),
        out_shape=jax.ShapeDtypeStruct((4, P, _HP), jnp.float32),
    )(pe_table.astype(jnp.float32), A.astype(jnp.float32),
      bq.astype(jnp.float32))
    return T.reshape(4 * P, _HP)


def _make_sc_lookup(n_rows, H, n_tab):
    per_w = n_rows // _NW
    nch = per_w // _CHUNK
    vec = H // _LANES
    cw = _CHUNK * H
    mesh = plsc.VectorSubcoreMesh(core_axis_name="c", subcore_axis_name="s")

    @functools.partial(
        pl.kernel,
        mesh=mesh,
        out_type=jax.ShapeDtypeStruct((n_rows * H,), jnp.float32),
        scratch_types=[
            pltpu.VMEM((2, 4, _CHUNK), jnp.int32),
            pltpu.VMEM((2, 4, _CHUNK, H), jnp.float32),
            pltpu.VMEM((2, cw), jnp.float32),
            pltpu.VMEM_SHARED((n_tab, H), jnp.float32),
            pltpu.SemaphoreType.DMA,
            pltpu.SemaphoreType.DMA,
            pltpu.SemaphoreType.DMA,
            pltpu.SemaphoreType.DMA,
            pltpu.SemaphoreType.DMA,
            pltpu.SemaphoreType.DMA,
        ],
        compiler_params=pltpu.CompilerParams(use_tc_tiling_on_sc=False),
    )
    def sc_fn(t_hbm, i0_hbm, i1_hbm, i2_hbm, i3_hbm, out_hbm,
              idx_v, rows_v, out_v, t_sp, si0, si1, sg0, sg1, ss0, ss1):
        wid = lax.axis_index("s") * _NC + lax.axis_index("c")
        base = wid * nch

        @pl.when(lax.axis_index("s") == 0)
        def _():
            pltpu.sync_copy(t_hbm.at[:, pl.ds(0, H)], t_sp)

        plsc.subcore_barrier()
        idx_refs = (i0_hbm, i1_hbm, i2_hbm, i3_hbm)
        sem_i = (si0, si1)
        sem_g = (sg0, sg1)
        sem_s = (ss0, ss1)

        def start_idx(ci, slot):
            off = (base + ci) * _CHUNK
            for k in range(4):
                pltpu.async_copy(idx_refs[k].at[pl.ds(off, _CHUNK)],
                                 idx_v.at[slot, k], sem_i[slot])

        def wait_idx(slot):
            pltpu.make_async_copy(i0_hbm.at[pl.ds(0, 4 * _CHUNK)],
                                  idx_v.at[slot], sem_i[slot]).wait()

        def start_gathers(buf):
            for k in range(4):
                pltpu.async_copy(t_sp.at[idx_v.at[buf, k]],
                                 rows_v.at[buf, k], sem_g[buf])

        def wait_gathers(buf):
            pltpu.make_async_copy(out_hbm.at[pl.ds(0, 4 * cw)],
                                  rows_v.at[buf], sem_g[buf]).wait()

        def wait_scatter(buf):
            pltpu.make_async_copy(out_hbm.at[pl.ds(0, cw)],
                                  out_v.at[buf], sem_s[buf]).wait()

        def compute_and_scatter(ci, buf):
            def row_body(r, c2):
                for v in range(vec):
                    sl = pl.ds(v * _LANES, _LANES)
                    acc = (rows_v[buf, 0, r, sl] + rows_v[buf, 1, r, sl]) + (
                        rows_v[buf, 2, r, sl] + rows_v[buf, 3, r, sl])
                    out_v[buf, pl.ds(r * H + v * _LANES, _LANES)] = (
                        jnp.maximum(acc, 0.0))
                return c2

            lax.fori_loop(0, _CHUNK, row_body, 0)
            pltpu.async_copy(out_v.at[buf],
                             out_hbm.at[pl.ds((base + ci) * cw, cw)],
                             sem_s[buf])

        def step(ci, buf, first):
            @pl.when(ci + 1 < nch)
            def _():
                wait_idx(1 - buf)
                start_gathers(1 - buf)
            wait_gathers(buf)

            @pl.when(ci + 2 < nch)
            def _():
                start_idx(ci + 2, buf)

            if not first:
                @pl.when(ci >= 2)
                def _():
                    wait_scatter(buf)
            compute_and_scatter(ci, buf)

        start_idx(0, 0)
        start_idx(1, 1)
        wait_idx(0)
        start_gathers(0)
        step(0, 0, True)

        def chunk_body(h, carry):
            step(1 + 2 * h, 1, False)
            step(2 + 2 * h, 0, False)
            return carry

        lax.fori_loop(0, (nch - 1) // 2, chunk_body, 0)
        if (nch - 1) % 2 == 1:
            step(nch - 1, (nch - 1) % 2, False)
        wait_scatter(0)
        wait_scatter(1)

    return sc_fn


def kernel(pos_s, pos_e, pe_table, W, b):
    B, L = pos_s.shape
    P, H = pe_table.shape
    M = P // 2
    n = B * L * L
    T = _make_tables(pe_table, W, b)
    ps = pos_s.astype(jnp.int32)
    pe = pos_e.astype(jnp.int32)

    def rel(a, c, off):
        d = jnp.clip(a[:, :, None] - c[:, None, :] + M, 0, P - 1)
        return (d + off).reshape(-1)

    out = _make_sc_lookup(n, H, 4 * P)(
        T, rel(ps, ps, 0), rel(ps, pe, P), rel(pe, ps, 2 * P),
        rel(pe, pe, 3 * P))
    return out.reshape(B, L, L, H)
